# baseline (speedup 1.0000x reference)
import functools

import jax
import jax.numpy as jnp
from jax import lax
from jax.experimental import pallas as pl
from jax.experimental.pallas import tpu as pltpu
from jax.experimental.pallas import tpu_sc as plsc

_B = 16384
_V = 10
_D = 32
_DP = 128

_cached = {}


def _make_kernel():
    if "k" in _cached:
        return _cached["k"]
    info = plsc.get_sparse_core_info()
    nc, ns, nl = info.num_cores, info.num_subcores, info.num_lanes
    nw = nc * ns
    b_per_w = _B // nw
    n_q = 4
    rows_q = b_per_w // n_q
    mesh = plsc.VectorSubcoreMesh(core_axis_name="c", subcore_axis_name="s")

    @functools.partial(
        pl.kernel,
        mesh=mesh,
        out_type=jax.ShapeDtypeStruct((_B, _D), jnp.float32),
        scratch_types=[
            pltpu.VMEM((b_per_w,), jnp.int32),
            pltpu.VMEM((b_per_w,), jnp.int32),
            pltpu.VMEM((_V, _D), jnp.float32),
            pltpu.VMEM((_V, _DP), jnp.float32),
            pltpu.VMEM_SHARED((ns * _V, _DP), jnp.float32),
            pltpu.VMEM((2, rows_q, _DP), jnp.float32),
            pltpu.VMEM((b_per_w, _D), jnp.float32),
            pltpu.SemaphoreType.DMA,
            pltpu.SemaphoreType.DMA,
            pltpu.SemaphoreType.DMA,
            pltpu.SemaphoreType.DMA,
            pltpu.SemaphoreType.DMA,
            pltpu.SemaphoreType.DMA,
        ],
        compiler_params=pltpu.CompilerParams(needs_layout_passes=False),
    )
    def k(t_hbm, table_hbm, out_hbm, idx_v, idxg_v, table_v, tpad_v, tpad_sh,
          rows_v, out_v, sem_i, sem_g0, sem_g1, sem_g2, sem_g3, sem_o):
        sid = lax.axis_index("s")
        wid = sid * nc + lax.axis_index("c")
        base = wid * b_per_w
        cp_i = pltpu.async_copy(t_hbm.at[pl.ds(base, b_per_w)], idx_v, sem_i)

        pltpu.sync_copy(table_hbm, table_v)
        for v in range(_V):
            tpad_v[v, pl.ds(0, nl)] = table_v[v, pl.ds(0, nl)]
            tpad_v[v, pl.ds(nl, nl)] = table_v[v, pl.ds(nl, nl)]
        pltpu.sync_copy(tpad_v, tpad_sh.at[pl.ds(sid * _V, _V)])

        cp_i.wait()
        slot = sid * _V
        for v in range(b_per_w // nl):
            idxg_v[pl.ds(v * nl, nl)] = idx_v[pl.ds(v * nl, nl)] + slot

        g_sems = [sem_g0, sem_g1, sem_g2, sem_g3]

        def fire_gather(q):
            return pltpu.async_copy(
                tpad_sh.at[idxg_v.at[pl.ds(q * rows_q, rows_q)]],
                rows_v.at[q % 2],
                g_sems[q],
            )

        cps_g = {0: fire_gather(0)}
        outcps = []
        for q in range(n_q):
            if q + 1 < n_q:
                cps_g[q + 1] = fire_gather(q + 1)
            cps_g[q].wait()
            buf = q % 2

            def body(i, carry, q=q, buf=buf):
                i0 = i * nl
                for j in range(nl):
                    r = q * rows_q + i0 + j
                    out_v[r, pl.ds(0, nl)] = rows_v[buf, i0 + j, pl.ds(0, nl)]
                    out_v[r, pl.ds(nl, nl)] = rows_v[buf, i0 + j, pl.ds(nl, nl)]
                return carry

            lax.fori_loop(0, rows_q // nl, body, 0)
            outcps.append(
                pltpu.async_copy(
                    out_v.at[pl.ds(q * rows_q, rows_q)],
                    out_hbm.at[pl.ds(base + q * rows_q, rows_q)],
                    sem_o,
                )
            )
        for cp in outcps:
            cp.wait()

    _cached["k"] = k
    return k


def kernel(t, table):
    k = _make_kernel()
    return k(t.astype(jnp.int32), table.astype(jnp.float32))


# speedup vs baseline: 1.0006x; 1.0006x over previous
import functools

import jax
import jax.numpy as jnp
from jax import lax
from jax.experimental import pallas as pl
from jax.experimental.pallas import tpu as pltpu
from jax.experimental.pallas import tpu_sc as plsc

_B = 16384
_V = 10
_D = 32
_DP = 128

_cached = {}


def _make_kernel():
    if "k" in _cached:
        return _cached["k"]
    info = plsc.get_sparse_core_info()
    nc, ns, nl = info.num_cores, info.num_subcores, info.num_lanes
    nw = nc * ns
    b_per_w = _B // nw
    n_q = 8
    rows_q = b_per_w // n_q
    mesh = plsc.VectorSubcoreMesh(core_axis_name="c", subcore_axis_name="s")

    @functools.partial(
        pl.kernel,
        mesh=mesh,
        out_type=jax.ShapeDtypeStruct((_B, _D), jnp.float32),
        scratch_types=[
            pltpu.VMEM((b_per_w,), jnp.int32),
            pltpu.VMEM((_V, _D), jnp.float32),
            pltpu.VMEM((_V, _DP), jnp.float32),
            pltpu.VMEM_SHARED((_V, _DP), jnp.float32),
            pltpu.VMEM((2, rows_q, _DP), jnp.float32),
            pltpu.VMEM((b_per_w, _D), jnp.float32),
            pltpu.SemaphoreType.DMA,
            pltpu.SemaphoreType.DMA,
            pltpu.SemaphoreType.DMA,
            pltpu.SemaphoreType.DMA,
            pltpu.SemaphoreType.DMA,
        ],
        compiler_params=pltpu.CompilerParams(needs_layout_passes=False),
    )
    def k(t_hbm, table_hbm, out_hbm, idx_v, table_v, tpad_v, tpad_sh, rows_v,
          out_v, sem_i, sem_t, sem_g0, sem_g1, sem_o):
        sid = lax.axis_index("s")
        wid = sid * nc + lax.axis_index("c")
        base = wid * b_per_w
        cp_i = pltpu.async_copy(t_hbm.at[pl.ds(base, b_per_w)], idx_v, sem_i)

        @pl.when(sid == 0)
        def _stage_table():
            pltpu.sync_copy(table_hbm, table_v)
            for v in range(_V):
                tpad_v[v, pl.ds(0, nl)] = table_v[v, pl.ds(0, nl)]
                tpad_v[v, pl.ds(nl, nl)] = table_v[v, pl.ds(nl, nl)]
            pltpu.sync_copy(tpad_v, tpad_sh)

        plsc.subcore_barrier()
        cp_i.wait()

        g_sems = [sem_g0, sem_g1]

        def fire_gather(q):
            return pltpu.async_copy(
                tpad_sh.at[idx_v.at[pl.ds(q * rows_q, rows_q)]],
                rows_v.at[q % 2],
                g_sems[q % 2],
            )

        cps_g = {0: fire_gather(0)}
        outcps = []
        for q in range(n_q):
            if q + 1 < n_q:
                cps_g[q + 1] = fire_gather(q + 1)
            cps_g[q].wait()
            buf = q % 2

            def body(i, carry, buf=buf, q=q):
                r0 = i * nl
                for j in range(nl):
                    r = r0 + j
                    out_v[q * rows_q + r, pl.ds(0, nl)] = rows_v[buf, r, pl.ds(0, nl)]
                    out_v[q * rows_q + r, pl.ds(nl, nl)] = rows_v[buf, r, pl.ds(nl, nl)]
                return carry

            lax.fori_loop(0, rows_q // nl, body, 0)
            outcps.append(
                pltpu.async_copy(
                    out_v.at[pl.ds(q * rows_q, rows_q)],
                    out_hbm.at[pl.ds(base + q * rows_q, rows_q)],
                    sem_o,
                )
            )
        for cp in outcps:
            cp.wait()

    _cached["k"] = k
    return k


def kernel(t, table):
    k = _make_kernel()
    return k(t.astype(jnp.int32), table.astype(jnp.float32))


# speedup vs baseline: 1.0015x; 1.0009x over previous
import functools

import jax
import jax.numpy as jnp
from jax import lax
from jax.experimental import pallas as pl
from jax.experimental.pallas import tpu as pltpu
from jax.experimental.pallas import tpu_sc as plsc

_B = 16384
_V = 10
_D = 32
_DP = 128

_cached = {}


def _make_kernel():
    if "k" in _cached:
        return _cached["k"]
    info = plsc.get_sparse_core_info()
    nc, ns, nl = info.num_cores, info.num_subcores, info.num_lanes
    nw = nc * ns
    b_per_w = _B // nw
    n_q = 4
    rows_q = b_per_w // n_q
    half = b_per_w // 2
    mesh = plsc.VectorSubcoreMesh(core_axis_name="c", subcore_axis_name="s")

    @functools.partial(
        pl.kernel,
        mesh=mesh,
        out_type=jax.ShapeDtypeStruct((_B, _D), jnp.float32),
        scratch_types=[
            pltpu.VMEM((b_per_w,), jnp.int32),
            pltpu.VMEM((_V, _D), jnp.float32),
            pltpu.VMEM((_V, _DP), jnp.float32),
            pltpu.VMEM_SHARED((_V, _DP), jnp.float32),
            pltpu.VMEM((2, rows_q, _DP), jnp.float32),
            pltpu.VMEM((b_per_w, _D), jnp.float32),
            pltpu.SemaphoreType.DMA,
            pltpu.SemaphoreType.DMA,
            pltpu.SemaphoreType.DMA,
            pltpu.SemaphoreType.DMA,
            pltpu.SemaphoreType.DMA,
        ],
        compiler_params=pltpu.CompilerParams(needs_layout_passes=False),
    )
    def k(t_hbm, table_hbm, out_hbm, idx_v, table_v, tpad_v, tpad_sh, rows_v,
          out_v, sem_i, sem_t, sem_g0, sem_g1, sem_o):
        sid = lax.axis_index("s")
        wid = sid * nc + lax.axis_index("c")
        base = wid * b_per_w
        cp_i = pltpu.async_copy(t_hbm.at[pl.ds(base, b_per_w)], idx_v, sem_i)
        pltpu.sync_copy(table_hbm, table_v)

        @pl.when(sid == 0)
        def _stage_table():
            for v in range(_V):
                tpad_v[v, pl.ds(0, nl)] = table_v[v, pl.ds(0, nl)]
                tpad_v[v, pl.ds(nl, nl)] = table_v[v, pl.ds(nl, nl)]
            pltpu.sync_copy(tpad_v, tpad_sh)

        plsc.subcore_barrier()
        cp_i.wait()

        g_sems = [sem_g0, sem_g1]

        def fire_gather(q):
            return pltpu.async_copy(
                tpad_sh.at[idx_v.at[pl.ds(q * rows_q, rows_q)]],
                rows_v.at[q % 2],
                g_sems[q % 2],
            )

        cps_g = {0: fire_gather(0), 1: fire_gather(1)}
        outcps = []

        def body2(i, carry):
            r0 = half + i * nl
            t_vec = idx_v[pl.ds(r0, nl)]
            for j in range(nl):
                r = r0 + j
                a = t_vec[j]
                out_v[r, pl.ds(0, nl)] = table_v[a, pl.ds(0, nl)]
                out_v[r, pl.ds(nl, nl)] = table_v[a, pl.ds(nl, nl)]
            return carry

        lax.fori_loop(0, half // nl, body2, 0)
        outcps.append(
            pltpu.async_copy(
                out_v.at[pl.ds(half, half)],
                out_hbm.at[pl.ds(base + half, half)],
                sem_o,
            )
        )
        for q in range(2):
            cps_g[q].wait()
            buf = q % 2

            def body(i, carry, buf=buf, q=q):
                r0 = i * nl
                for j in range(nl):
                    r = r0 + j
                    out_v[q * rows_q + r, pl.ds(0, nl)] = rows_v[buf, r, pl.ds(0, nl)]
                    out_v[q * rows_q + r, pl.ds(nl, nl)] = rows_v[buf, r, pl.ds(nl, nl)]
                return carry

            lax.fori_loop(0, rows_q // nl, body, 0)
            outcps.append(
                pltpu.async_copy(
                    out_v.at[pl.ds(q * rows_q, rows_q)],
                    out_hbm.at[pl.ds(base + q * rows_q, rows_q)],
                    sem_o,
                )
            )
        for cp in outcps:
            cp.wait()

    _cached["k"] = k
    return k


def kernel(t, table):
    k = _make_kernel()
    return k(t.astype(jnp.int32), table.astype(jnp.float32))


# speedup vs baseline: 1.0337x; 1.0321x over previous
import functools

import jax
import jax.numpy as jnp
from jax import lax
from jax.experimental import pallas as pl
from jax.experimental.pallas import tpu as pltpu
from jax.experimental.pallas import tpu_sc as plsc

_B = 16384
_V = 10
_D = 32
_DP = 128

_cached = {}


def _make_kernel():
    if "k" in _cached:
        return _cached["k"]
    info = plsc.get_sparse_core_info()
    nc, ns, nl = info.num_cores, info.num_subcores, info.num_lanes
    nw = nc * ns
    b_per_w = _B // nw
    n_q = 4
    rows_q = b_per_w // n_q
    mesh = plsc.VectorSubcoreMesh(core_axis_name="c", subcore_axis_name="s")

    @functools.partial(
        pl.kernel,
        mesh=mesh,
        out_type=jax.ShapeDtypeStruct((_B, _D), jnp.float32),
        scratch_types=[
            pltpu.VMEM((b_per_w,), jnp.int32),
            pltpu.VMEM((_V, _D), jnp.float32),
            pltpu.VMEM((_V, _DP), jnp.float32),
            pltpu.VMEM_SHARED((_V, _DP), jnp.float32),
            pltpu.VMEM((2, rows_q, _DP), jnp.float32),
            pltpu.VMEM((b_per_w, _D), jnp.float32),
            pltpu.SemaphoreType.DMA,
            pltpu.SemaphoreType.DMA,
            pltpu.SemaphoreType.DMA,
            pltpu.SemaphoreType.DMA,
            pltpu.SemaphoreType.DMA,
        ],
        compiler_params=pltpu.CompilerParams(needs_layout_passes=False),
    )
    def k(t_hbm, table_hbm, out_hbm, idx_v, table_v, tpad_v, tpad_sh, rows_v,
          out_v, sem_i, sem_t, sem_g0, sem_g1, sem_o):
        sid = lax.axis_index("s")
        wid = sid * nc + lax.axis_index("c")
        base = wid * b_per_w
        cp_i = pltpu.async_copy(t_hbm.at[pl.ds(base, b_per_w)], idx_v, sem_i)

        @pl.when(sid == 0)
        def _stage_table():
            pltpu.sync_copy(table_hbm, table_v)
            for v in range(_V):
                tpad_v[v, pl.ds(0, nl)] = table_v[v, pl.ds(0, nl)]
                tpad_v[v, pl.ds(nl, nl)] = table_v[v, pl.ds(nl, nl)]
            pltpu.sync_copy(tpad_v, tpad_sh)

        plsc.subcore_barrier()
        cp_i.wait()

        g_sems = [sem_g0, sem_g1]

        def fire_gather(q):
            return pltpu.async_copy(
                tpad_sh.at[idx_v.at[pl.ds(q * rows_q, rows_q)]],
                rows_v.at[q % 2],
                g_sems[q % 2],
            )

        cps_g = {0: fire_gather(0)}
        outcps = []
        for q in range(n_q):
            if q + 1 < n_q:
                cps_g[q + 1] = fire_gather(q + 1)
            cps_g[q].wait()
            buf = q % 2

            def body(i, carry, buf=buf, q=q):
                r0 = i * nl
                for j in range(nl):
                    r = r0 + j
                    out_v[q * rows_q + r, pl.ds(0, nl)] = rows_v[buf, r, pl.ds(0, nl)]
                    out_v[q * rows_q + r, pl.ds(nl, nl)] = rows_v[buf, r, pl.ds(nl, nl)]
                return carry

            lax.fori_loop(0, rows_q // nl, body, 0)
            outcps.append(
                pltpu.async_copy(
                    out_v.at[pl.ds(q * rows_q, rows_q)],
                    out_hbm.at[pl.ds(base + q * rows_q, rows_q)],
                    sem_o,
                )
            )
        for cp in outcps:
            cp.wait()

    _cached["k"] = k
    return k


def kernel(t, table):
    k = _make_kernel()
    return k(t.astype(jnp.int32), table.astype(jnp.float32))
